# baseline (speedup 1.0000x reference)
import functools
import jax
import jax.numpy as jnp
from jax import lax
from jax.experimental import pallas as pl
from jax.experimental.pallas import tpu as pltpu
from jax.experimental.pallas import tpu_sc as plsc

_L = 16


def _sc_body(dims, row_hbm, col_hbm, dep_hbm, out_hbm,
             rowv, depv, slab0, slab1, sem0, sem1):
    d, h, w, c, nw = dims
    ypw = h // nw
    cid = lax.axis_index("c")
    sid = lax.axis_index("s")
    wid = sid * 2 + cid
    y0 = wid * ypw

    pltpu.sync_copy(row_hbm.at[pl.ds(0, h)], rowv)
    pltpu.sync_copy(dep_hbm.at[pl.ds(0, d)], depv)
    pltpu.sync_copy(col_hbm.at[pl.ds(0, w)], slab0.at[:, pl.ds(0, 64)])
    pltpu.sync_copy(col_hbm.at[pl.ds(0, w)], slab1.at[:, pl.ds(0, 64)])

    slabs = (slab0, slab1)
    sems = (sem0, sem1)
    pend = [None, None]

    def fill(buf, off, regs):
        def body(x, carry):
            for i, r in enumerate(regs):
                buf[x, pl.ds(off + i * _L, _L)] = r
            return carry
        lax.fori_loop(0, w, body, 0)

    for z in range(d):
        rz = [depv[z, pl.ds(i * _L, _L)] for i in range(4)]
        for b in range(2):
            if pend[b] is not None:
                pend[b].wait()
                pend[b] = None
            fill(slabs[b], 128, rz)
        for yi in range(ypw):
            b = yi & 1
            y = y0 + yi
            if pend[b] is not None:
                pend[b].wait()
                pend[b] = None
            ry = [rowv[y, pl.ds(i * _L, _L)] for i in range(4)]
            fill(slabs[b], 64, ry)
            pend[b] = pltpu.async_copy(slabs[b], out_hbm.at[z, y], sems[b])
    for b in range(2):
        if pend[b] is not None:
            pend[b].wait()


def kernel(scan, row_weight, col_weight, depth_weight):
    d, em, h, w = scan.shape
    c = row_weight.shape[1] + col_weight.shape[1] + depth_weight.shape[1]
    nw = 32
    mesh = plsc.VectorSubcoreMesh(core_axis_name="c", subcore_axis_name="s")
    body = functools.partial(_sc_body, (d, h, w, c, nw))
    k = pl.kernel(
        body,
        out_type=jax.ShapeDtypeStruct((d, h, w, c), jnp.float32),
        mesh=mesh,
        scratch_types=[
            pltpu.VMEM((h, 64), jnp.float32),
            pltpu.VMEM((d, 64), jnp.float32),
            pltpu.VMEM((w, c), jnp.float32),
            pltpu.VMEM((w, c), jnp.float32),
            pltpu.SemaphoreType.DMA,
            pltpu.SemaphoreType.DMA,
        ],
        compiler_params=pltpu.CompilerParams(use_tc_tiling_on_sc=False),
    )
    return k(row_weight, col_weight, depth_weight)


# speedup vs baseline: 3.9431x; 3.9431x over previous
import jax
import jax.numpy as jnp
from jax.experimental import pallas as pl
from jax.experimental.pallas import tpu as pltpu

_BH = 32
_K = 4
_RB = _BH // _K


def _make_body(d, h, w, c):
    nh = h // _BH
    steps = d * nh

    def body(row_ref, col_ref, dep_ref, out_hbm, scratch, sems):
        s = pl.program_id(0)
        b = jax.lax.rem(s, 2)
        di = jax.lax.div(s, nh)
        hi = jax.lax.rem(s, nh)

        def wait_for(ps):
            pb = jax.lax.rem(ps, 2)
            pdi = jax.lax.div(ps, nh)
            phi = jax.lax.rem(ps, nh)
            for k in range(_K):
                pltpu.make_async_copy(
                    scratch.at[pb, pl.ds(k * _RB, _RB)],
                    out_hbm.at[pdi, pl.ds(phi * _BH + k * _RB, _RB)],
                    sems.at[pb, k],
                ).wait()

        @pl.when(s >= 2)
        def _():
            wait_for(s - 2)

        x = col_ref[:w, :]
        y = row_ref[pl.ds(hi * _BH, _BH), :]
        z = dep_ref[pl.ds(di, 1), :]
        xb = jnp.broadcast_to(x[None, :, :], (_BH, w, 64))
        yb = jnp.broadcast_to(y[:, None, :], (_BH, w, 64))
        zb = jnp.broadcast_to(z[:, None, :], (_BH, w, 64))
        scratch[b] = jnp.concatenate([xb, yb, zb], axis=-1)

        for k in range(_K):
            pltpu.make_async_copy(
                scratch.at[b, pl.ds(k * _RB, _RB)],
                out_hbm.at[di, pl.ds(hi * _BH + k * _RB, _RB)],
                sems.at[b, k],
            ).start()

        @pl.when(s == steps - 1)
        def _():
            wait_for(s - 1)
            wait_for(s)

    return body


def kernel(scan, row_weight, col_weight, depth_weight):
    d, em, h, w = scan.shape
    c = row_weight.shape[1] + col_weight.shape[1] + depth_weight.shape[1]
    nh = h // _BH
    return pl.pallas_call(
        _make_body(d, h, w, c),
        grid=(d * nh,),
        in_specs=[
            pl.BlockSpec(row_weight.shape, lambda s: (0, 0)),
            pl.BlockSpec(col_weight.shape, lambda s: (0, 0)),
            pl.BlockSpec(depth_weight.shape, lambda s: (0, 0)),
        ],
        out_specs=pl.BlockSpec(memory_space=pl.ANY),
        out_shape=jax.ShapeDtypeStruct((d, h, w, c), jnp.float32),
        scratch_shapes=[
            pltpu.VMEM((2, _BH, w, c), jnp.float32),
            pltpu.SemaphoreType.DMA((2, _K)),
        ],
        compiler_params=pltpu.CompilerParams(
            dimension_semantics=("arbitrary",),
        ),
    )(row_weight, col_weight, depth_weight)
